# Initial kernel scaffold; baseline (speedup 1.0000x reference)
import jax
import jax.numpy as jnp
from jax.experimental import pallas as pl


def kernel(x, edge_index, W_pre, b_pre, Wl1, bl1, Wr1, Wl2, bl2, Wr2, Wl3, bl3, Wr3):
    raise NotImplementedError("write your pallas kernel here")


# speedup vs baseline: 2.8679x; 2.8679x over previous
import functools

import jax
import jax.numpy as jnp
from jax import lax
from jax.experimental import pallas as pl
from jax.experimental.pallas import tpu as pltpu
from jax.experimental.pallas import tpu_sc as plsc

N = 10000
D = 128
DH = D // 2
E = 320000

NC = 2
NS = 16
NW = NC * NS
CHUNK = 128
CPT = 80
EPT = CPT * CHUNK
E_PAD = NW * EPT
N_PAD = 10240
RPT = N_PAD // NS
KPT = RPT // CHUNK
TRASH = N
CW = 16

_MESH = plsc.VectorSubcoreMesh(core_axis_name="c", subcore_axis_name="s")


def _seg_sum_body(with_cnt, h_hbm, src_hbm, dst_hbm, *rest):
    if with_cnt:
        (out_p, out_c, src_v, dst_v, rows0, rows1, acc, sem0, sem1,
         ones_v, accc) = rest
    else:
        out_p, src_v, dst_v, rows0, rows1, acc, sem0, sem1 = rest
    c = lax.axis_index("c")
    s = lax.axis_index("s")
    wid = c * NS + s

    pltpu.sync_copy(src_hbm.at[wid], src_v)
    pltpu.sync_copy(dst_hbm.at[wid], dst_v)

    @pl.loop(0, CHUNK)
    def _(r):
        @pl.loop(0, DH, step=16)
        def _(cc):
            rows0.at[r, pl.ds(cc, 16)][...] = jnp.zeros((16,), jnp.float32)

    for k in range(KPT):
        pltpu.sync_copy(rows0, acc.at[pl.ds(s * RPT + k * CHUNK, CHUNK)])

    if with_cnt:
        @pl.loop(0, CHUNK)
        def _(r):
            ones_v.at[r, pl.ds(0, CW)][...] = jnp.zeros((CW,), jnp.float32)

        for k in range(KPT):
            pltpu.sync_copy(ones_v, accc.at[pl.ds(s * RPT + k * CHUNK, CHUNK)])

        @pl.loop(0, CHUNK)
        def _(r):
            ones_v.at[r, pl.ds(0, CW)][...] = jnp.ones((CW,), jnp.float32)

    plsc.subcore_barrier()

    pltpu.async_copy(h_hbm.at[src_v.at[0]], rows0, sem0)

    @pl.loop(0, CPT, step=2)
    def _(g):
        pltpu.make_async_copy(h_hbm.at[src_v.at[g]], rows0, sem0).wait()
        pltpu.async_copy(h_hbm.at[src_v.at[g + 1]], rows1, sem1)
        pltpu.sync_copy(rows0, acc.at[dst_v.at[g]], add=True)
        if with_cnt:
            pltpu.sync_copy(ones_v, accc.at[dst_v.at[g]], add=True)
        pltpu.make_async_copy(h_hbm.at[src_v.at[g + 1]], rows1, sem1).wait()

        @pl.when(g + 2 < CPT)
        def _():
            pltpu.async_copy(h_hbm.at[src_v.at[g + 2]], rows0, sem0)

        pltpu.sync_copy(rows1, acc.at[dst_v.at[g + 1]], add=True)
        if with_cnt:
            pltpu.sync_copy(ones_v, accc.at[dst_v.at[g + 1]], add=True)

    plsc.subcore_barrier()

    for k in range(KPT):
        rs = s * RPT + k * CHUNK
        pltpu.sync_copy(acc.at[pl.ds(rs, CHUNK)],
                        out_p.at[c, pl.ds(rs, CHUNK)])
    if with_cnt:
        for k in range(KPT):
            rs = s * RPT + k * CHUNK
            pltpu.sync_copy(accc.at[pl.ds(rs, CHUNK)],
                            out_c.at[c, pl.ds(rs, CHUNK)])


def _seg_sum(h2, src3, dst3, with_cnt):
    outs = [jax.ShapeDtypeStruct((NC, N_PAD, DH), jnp.float32)]
    scratch = [
        pltpu.VMEM((CPT, CHUNK), jnp.int32),
        pltpu.VMEM((CPT, CHUNK), jnp.int32),
        pltpu.VMEM((CHUNK, DH), jnp.float32),
        pltpu.VMEM((CHUNK, DH), jnp.float32),
        pltpu.VMEM_SHARED((N_PAD, DH), jnp.float32),
        pltpu.SemaphoreType.DMA,
        pltpu.SemaphoreType.DMA,
    ]
    if with_cnt:
        outs.append(jax.ShapeDtypeStruct((NC, N_PAD, CW), jnp.float32))
        scratch += [
            pltpu.VMEM((CHUNK, CW), jnp.float32),
            pltpu.VMEM_SHARED((N_PAD, CW), jnp.float32),
        ]
    fn = pl.kernel(
        functools.partial(_seg_sum_body, with_cnt),
        out_type=tuple(outs),
        mesh=_MESH,
        scratch_types=scratch,
        compiler_params=pltpu.CompilerParams(use_tc_tiling_on_sc=False),
    )
    return fn(h2, src3, dst3)


def _linear_body(x_ref, w_ref, b_ref, o_ref):
    o_ref[...] = lax.dot_general(
        x_ref[...], w_ref[...], (((1,), (1,)), ((), ())),
        preferred_element_type=jnp.float32) + b_ref[...]


_RB = 400
_GRID = N // _RB


def _linear(x, w, b):
    return pl.pallas_call(
        _linear_body,
        grid=(_GRID,),
        in_specs=[
            pl.BlockSpec((_RB, D), lambda i: (i, 0)),
            pl.BlockSpec((D, D), lambda i: (0, 0)),
            pl.BlockSpec((1, D), lambda i: (0, 0)),
        ],
        out_specs=pl.BlockSpec((_RB, D), lambda i: (i, 0)),
        out_shape=jax.ShapeDtypeStruct((N, D), jnp.float32),
    )(x, w, b.reshape(1, D))


def _sage_body(mode, pa_ref, pb_ref, c_ref, h_ref, wl_ref, bl_ref, wr_ref,
               o_ref):
    cnt = jnp.maximum(c_ref[0][:, 0:1] + c_ref[1][:, 0:1], 1.0)
    mean = jnp.concatenate(
        [pa_ref[0] + pa_ref[1], pb_ref[0] + pb_ref[1]], axis=1) / cnt
    z = lax.dot_general(mean, wl_ref[...], (((1,), (1,)), ((), ())),
                        preferred_element_type=jnp.float32)
    z = z + lax.dot_general(h_ref[...], wr_ref[...], (((1,), (1,)), ((), ())),
                            preferred_element_type=jnp.float32)
    z = z + bl_ref[...]
    if mode == "relu":
        z = jnp.maximum(z, 0.0)
    elif mode == "norm":
        nrm = jnp.sqrt(jnp.sum(z * z, axis=1, keepdims=True))
        z = z / jnp.maximum(nrm, 1e-12)
    o_ref[...] = z


def _sage_tc(pa, pb, cnt, h, wl, bl, wr, mode):
    return pl.pallas_call(
        functools.partial(_sage_body, mode),
        grid=(_GRID,),
        in_specs=[
            pl.BlockSpec((NC, _RB, DH), lambda i: (0, i, 0)),
            pl.BlockSpec((NC, _RB, DH), lambda i: (0, i, 0)),
            pl.BlockSpec((NC, _RB, CW), lambda i: (0, i, 0)),
            pl.BlockSpec((_RB, D), lambda i: (i, 0)),
            pl.BlockSpec((D, D), lambda i: (0, 0)),
            pl.BlockSpec((1, D), lambda i: (0, 0)),
            pl.BlockSpec((D, D), lambda i: (0, 0)),
        ],
        out_specs=pl.BlockSpec((_RB, D), lambda i: (i, 0)),
        out_shape=jax.ShapeDtypeStruct((N, D), jnp.float32),
    )(pa, pb, cnt, h, wl, bl.reshape(1, D), wr)


def _layer(h, srcA3, srcB3, dst3, cnt, wl, bl, wr, mode):
    h2 = h.reshape(2 * N, DH)
    pa, = _seg_sum(h2, srcA3, dst3, with_cnt=False)
    pb, = _seg_sum(h2, srcB3, dst3, with_cnt=False)
    return _sage_tc(pa, pb, cnt, h, wl, bl, wr, mode)


def kernel(x, edge_index, W_pre, b_pre, Wl1, bl1, Wr1, Wl2, bl2, Wr2,
           Wl3, bl3, Wr3):
    src = edge_index[0].astype(jnp.int32)
    dst = edge_index[1].astype(jnp.int32)
    pad = E_PAD - E
    src_pad = jnp.concatenate([src, jnp.zeros((pad,), jnp.int32)])
    srcA3 = (2 * src_pad).reshape(NW, CPT, CHUNK)
    srcB3 = (2 * src_pad + 1).reshape(NW, CPT, CHUNK)
    dst3 = jnp.concatenate(
        [dst, jnp.full((pad,), TRASH, jnp.int32)]).reshape(NW, CPT, CHUNK)

    h0 = _linear(x, W_pre, b_pre)
    h02 = h0.reshape(2 * N, DH)
    pa1, cnt = _seg_sum(h02, srcA3, dst3, with_cnt=True)
    pb1, = _seg_sum(h02, srcB3, dst3, with_cnt=False)
    h1 = _sage_tc(pa1, pb1, cnt, h0, Wl1, bl1, Wr1, "relu")
    h2 = _layer(h1, srcA3, srcB3, dst3, cnt, Wl2, bl2, Wr2, "relu")
    return _layer(h2, srcA3, srcB3, dst3, cnt, Wl3, bl3, Wr3, "norm")


# speedup vs baseline: 4.1258x; 1.4386x over previous
import functools

import jax
import jax.numpy as jnp
from jax import lax
from jax.experimental import pallas as pl
from jax.experimental.pallas import tpu as pltpu
from jax.experimental.pallas import tpu_sc as plsc

N = 10000
D = 128
DH = D // 2
E = 320000

NC = 2
NS = 16
CHUNK = 128
CPT = 160
EPT = CPT * CHUNK
E_PAD = NS * EPT
N_PAD = 10240
RPT = N_PAD // NS
KPT = RPT // CHUNK
TRASH = N
CW = 16
NBUF = 4
DEPTH = NBUF // 2

_MESH = plsc.VectorSubcoreMesh(core_axis_name="c", subcore_axis_name="s")


def _seg_sum_body(with_cnt, h_hbm, src_hbm, dst_hbm, *rest):
    if with_cnt:
        (out_p, out_c, src_v, dst_v, rows, sem_g, sem_s, acc,
         ones_v, sem_c, accc) = rest
    else:
        out_p, src_v, dst_v, rows, sem_g, sem_s, acc = rest
    c = lax.axis_index("c")
    s = lax.axis_index("s")

    pltpu.sync_copy(src_hbm.at[c, s], src_v)
    pltpu.sync_copy(dst_hbm.at[s], dst_v)

    @pl.loop(0, CHUNK)
    def _(r):
        @pl.loop(0, DH, step=16)
        def _(cc):
            rows.at[0, r, pl.ds(cc, 16)][...] = jnp.zeros((16,), jnp.float32)

    for k in range(KPT):
        pltpu.sync_copy(rows.at[0],
                        acc.at[pl.ds(s * RPT + k * CHUNK, CHUNK)])

    if with_cnt:
        @pl.loop(0, CHUNK)
        def _(r):
            ones_v.at[r, pl.ds(0, CW)][...] = jnp.zeros((CW,), jnp.float32)

        for k in range(KPT):
            pltpu.sync_copy(
                ones_v, accc.at[pl.ds(s * RPT + k * CHUNK, CHUNK)])

        @pl.loop(0, CHUNK)
        def _(r):
            ones_v.at[r, pl.ds(0, CW)][...] = jnp.ones((CW,), jnp.float32)

    plsc.subcore_barrier()

    def gather_start(j, b):
        pltpu.async_copy(h_hbm.at[src_v.at[j]], rows.at[b], sem_g)

    def gather_wait(j, b):
        pltpu.make_async_copy(h_hbm.at[src_v.at[j]], rows.at[b], sem_g).wait()

    def scat_start(j, b):
        pltpu.async_copy(rows.at[b], acc.at[dst_v.at[j]], sem_s, add=True)

    def scat_wait(j, b):
        pltpu.make_async_copy(rows.at[b], acc.at[dst_v.at[j]], sem_s).wait()

    def cnt_start(j):
        pltpu.async_copy(ones_v, accc.at[dst_v.at[j]], sem_c, add=True)

    def cnt_wait(j):
        pltpu.make_async_copy(ones_v, accc.at[dst_v.at[j]], sem_c).wait()

    for b in range(DEPTH):
        gather_start(b, b)

    @pl.loop(0, CPT, step=NBUF)
    def _(g):
        for b in range(NBUF):
            j = g + b
            gather_wait(j, b % NBUF)
            scat_start(j, b % NBUF)
            if with_cnt:
                cnt_start(j)

                @pl.when(j >= DEPTH)
                def _():
                    cnt_wait(j)

            @pl.when(j >= DEPTH)
            def _():
                scat_wait(j - DEPTH, (b - DEPTH) % NBUF)

            @pl.when(j + DEPTH < CPT)
            def _():
                gather_start(j + DEPTH, (b + DEPTH) % NBUF)

    for b in range(DEPTH):
        scat_wait(CPT - DEPTH + b, (CPT - DEPTH + b) % NBUF)
        if with_cnt:
            cnt_wait(CPT - DEPTH + b)

    plsc.subcore_barrier()

    pltpu.sync_copy(acc.at[pl.ds(s * RPT, RPT)],
                    out_p.at[c].at[pl.ds(s * RPT, RPT)])
    if with_cnt:
        pltpu.sync_copy(accc.at[pl.ds(s * RPT, RPT)],
                        out_c.at[c].at[pl.ds(s * RPT, RPT)])


def _seg_sum(h2, src4, dst3, with_cnt):
    outs = [jax.ShapeDtypeStruct((NC, N_PAD, DH), jnp.float32)]
    scratch = [
        pltpu.VMEM((CPT, CHUNK), jnp.int32),
        pltpu.VMEM((CPT, CHUNK), jnp.int32),
        pltpu.VMEM((NBUF, CHUNK, DH), jnp.float32),
        pltpu.SemaphoreType.DMA,
        pltpu.SemaphoreType.DMA,
        pltpu.VMEM_SHARED((N_PAD, DH), jnp.float32),
    ]
    if with_cnt:
        outs.append(jax.ShapeDtypeStruct((NC, N_PAD, CW), jnp.float32))
        scratch += [
            pltpu.VMEM((CHUNK, CW), jnp.float32),
            pltpu.SemaphoreType.DMA,
            pltpu.VMEM_SHARED((N_PAD, CW), jnp.float32),
        ]
    fn = pl.kernel(
        functools.partial(_seg_sum_body, with_cnt),
        out_type=tuple(outs),
        mesh=_MESH,
        scratch_types=scratch,
        compiler_params=pltpu.CompilerParams(use_tc_tiling_on_sc=False),
    )
    return fn(h2, src4, dst3)


def _linear_body(x_ref, w_ref, b_ref, o_ref):
    o_ref[...] = lax.dot_general(
        x_ref[...], w_ref[...], (((1,), (1,)), ((), ())),
        preferred_element_type=jnp.float32) + b_ref[...]


_RB = 400
_GRID = N // _RB


def _linear(x, w, b):
    return pl.pallas_call(
        _linear_body,
        grid=(_GRID,),
        in_specs=[
            pl.BlockSpec((_RB, D), lambda i: (i, 0)),
            pl.BlockSpec((D, D), lambda i: (0, 0)),
            pl.BlockSpec((1, D), lambda i: (0, 0)),
        ],
        out_specs=pl.BlockSpec((_RB, D), lambda i: (i, 0)),
        out_shape=jax.ShapeDtypeStruct((N, D), jnp.float32),
    )(x, w, b.reshape(1, D))


def _sage_body(mode, p_ref, c_ref, h_ref, wl_ref, bl_ref, wr_ref, o_ref):
    cnt = jnp.maximum(c_ref[0][:, 0:1], 1.0)
    mean = jnp.concatenate([p_ref[0], p_ref[1]], axis=1) / cnt
    z = lax.dot_general(mean, wl_ref[...], (((1,), (1,)), ((), ())),
                        preferred_element_type=jnp.float32)
    z = z + lax.dot_general(h_ref[...], wr_ref[...], (((1,), (1,)), ((), ())),
                            preferred_element_type=jnp.float32)
    z = z + bl_ref[...]
    if mode == "relu":
        z = jnp.maximum(z, 0.0)
    elif mode == "norm":
        nrm = jnp.sqrt(jnp.sum(z * z, axis=1, keepdims=True))
        z = z / jnp.maximum(nrm, 1e-12)
    o_ref[...] = z


def _sage_tc(p, cnt, h, wl, bl, wr, mode):
    return pl.pallas_call(
        functools.partial(_sage_body, mode),
        grid=(_GRID,),
        in_specs=[
            pl.BlockSpec((NC, _RB, DH), lambda i: (0, i, 0)),
            pl.BlockSpec((NC, _RB, CW), lambda i: (0, i, 0)),
            pl.BlockSpec((_RB, D), lambda i: (i, 0)),
            pl.BlockSpec((D, D), lambda i: (0, 0)),
            pl.BlockSpec((1, D), lambda i: (0, 0)),
            pl.BlockSpec((D, D), lambda i: (0, 0)),
        ],
        out_specs=pl.BlockSpec((_RB, D), lambda i: (i, 0)),
        out_shape=jax.ShapeDtypeStruct((N, D), jnp.float32),
    )(p, cnt, h, wl, bl.reshape(1, D), wr)


def kernel(x, edge_index, W_pre, b_pre, Wl1, bl1, Wr1, Wl2, bl2, Wr2,
           Wl3, bl3, Wr3):
    src = edge_index[0].astype(jnp.int32)
    dst = edge_index[1].astype(jnp.int32)
    pad = E_PAD - E
    src_pad = jnp.concatenate([src, jnp.zeros((pad,), jnp.int32)])
    sp3 = src_pad.reshape(NS, CPT, CHUNK)
    src4 = jnp.stack([2 * sp3, 2 * sp3 + 1])
    dst3 = jnp.concatenate(
        [dst, jnp.full((pad,), TRASH, jnp.int32)]).reshape(NS, CPT, CHUNK)

    h0 = _linear(x, W_pre, b_pre)
    p1, cnt = _seg_sum(h0.reshape(2 * N, DH), src4, dst3, with_cnt=True)
    h1 = _sage_tc(p1, cnt, h0, Wl1, bl1, Wr1, "relu")
    p2, = _seg_sum(h1.reshape(2 * N, DH), src4, dst3, with_cnt=False)
    h2 = _sage_tc(p2, cnt, h1, Wl2, bl2, Wr2, "relu")
    p3, = _seg_sum(h2.reshape(2 * N, DH), src4, dst3, with_cnt=False)
    return _sage_tc(p3, cnt, h2, Wl3, bl3, Wr3, "norm")


# speedup vs baseline: 8.9835x; 2.1774x over previous
import functools

import jax
import jax.numpy as jnp
from jax import lax
from jax.experimental import pallas as pl
from jax.experimental.pallas import tpu as pltpu
from jax.experimental.pallas import tpu_sc as plsc

N = 10000
D = 128
DH = D // 2
E = 320000

NC = 2
NS = 16
CHUNK = 128
CPT = 160
EPT = CPT * CHUNK
E_PAD = NS * EPT
N_PAD = 10240
RPT = N_PAD // NS
KPT = RPT // CHUNK
HPT = N // NS
TRASH = N
CW = 16
NBUF = 4
DEPTH = NBUF // 2
NIDX = 8

_MESH = plsc.VectorSubcoreMesh(core_axis_name="c", subcore_axis_name="s")


def _seg_sum_body(with_cnt, hs_hbm, idx_hbm, *rest):
    if with_cnt:
        (out_p, out_c, sidx, rows, sem_i, sem_g, sem_s, acc, h_sp,
         ones_v, sem_c, accc) = rest
    else:
        out_p, sidx, rows, sem_i, sem_g, sem_s, acc, h_sp = rest
    c = lax.axis_index("c")
    s = lax.axis_index("s")

    pltpu.sync_copy(hs_hbm.at[c].at[pl.ds(s * HPT, HPT)],
                    h_sp.at[pl.ds(s * HPT, HPT)])

    @pl.loop(0, CHUNK)
    def _(r):
        @pl.loop(0, DH, step=16)
        def _(cc):
            rows.at[0, r, pl.ds(cc, 16)][...] = jnp.zeros((16,), jnp.float32)

    for k in range(KPT):
        pltpu.sync_copy(rows.at[0],
                        acc.at[pl.ds(s * RPT + k * CHUNK, CHUNK)])

    if with_cnt:
        @pl.loop(0, CHUNK)
        def _(r):
            ones_v.at[r, pl.ds(0, CW)][...] = jnp.zeros((CW,), jnp.float32)

        for k in range(KPT):
            pltpu.sync_copy(
                ones_v, accc.at[pl.ds(s * RPT + k * CHUNK, CHUNK)])

        @pl.loop(0, CHUNK)
        def _(r):
            ones_v.at[r, pl.ds(0, CW)][...] = jnp.ones((CW,), jnp.float32)

    plsc.subcore_barrier()

    def idx_start(j, ib):
        pltpu.async_copy(idx_hbm.at[s].at[j], sidx.at[ib], sem_i)

    def idx_wait(j, ib):
        pltpu.make_async_copy(idx_hbm.at[s].at[j], sidx.at[ib], sem_i).wait()

    def gather_start(ib, rb):
        pltpu.async_copy(h_sp.at[sidx.at[ib].at[0]], rows.at[rb], sem_g)

    def gather_wait(ib, rb):
        pltpu.make_async_copy(
            h_sp.at[sidx.at[ib].at[0]], rows.at[rb], sem_g).wait()

    def scat_start(ib, rb):
        pltpu.async_copy(rows.at[rb], acc.at[sidx.at[ib].at[1]], sem_s,
                         add=True)

    def scat_wait(ib, rb):
        pltpu.make_async_copy(
            rows.at[rb], acc.at[sidx.at[ib].at[1]], sem_s).wait()

    def cnt_start(ib):
        pltpu.async_copy(ones_v, accc.at[sidx.at[ib].at[1]], sem_c, add=True)

    def cnt_wait(ib):
        pltpu.make_async_copy(ones_v, accc.at[sidx.at[ib].at[1]], sem_c).wait()

    for t in range(DEPTH + 1):
        idx_start(t, t % NIDX)
    for b in range(DEPTH):
        idx_wait(b, b % NIDX)
        gather_start(b % NIDX, b % NBUF)

    @pl.loop(0, CPT, step=NIDX)
    def _(g):
        for b in range(NIDX):
            j = g + b
            gather_wait(b % NIDX, b % NBUF)
            scat_start(b % NIDX, b % NBUF)
            if with_cnt:
                cnt_start(b % NIDX)

                @pl.when(j >= DEPTH)
                def _():
                    cnt_wait(b % NIDX)

            @pl.when(j >= DEPTH)
            def _():
                scat_wait((b - DEPTH) % NIDX, (b - DEPTH) % NBUF)

            @pl.when(j + DEPTH < CPT)
            def _():
                idx_wait(j + DEPTH, (b + DEPTH) % NIDX)
                gather_start((b + DEPTH) % NIDX, (b + DEPTH) % NBUF)

            @pl.when(j + DEPTH + 1 < CPT)
            def _():
                idx_start(j + DEPTH + 1, (b + DEPTH + 1) % NIDX)

    for b in range(DEPTH):
        j = CPT - DEPTH + b
        scat_wait(j % NIDX, j % NBUF)
        if with_cnt:
            cnt_wait(j % NIDX)

    plsc.subcore_barrier()

    pltpu.sync_copy(acc.at[pl.ds(s * RPT, RPT)],
                    out_p.at[c].at[pl.ds(s * RPT, RPT)])
    if with_cnt:
        pltpu.sync_copy(accc.at[pl.ds(s * RPT, RPT)],
                        out_c.at[c].at[pl.ds(s * RPT, RPT)])


def _seg_sum(hs, idx, with_cnt):
    outs = [jax.ShapeDtypeStruct((NC, N_PAD, DH), jnp.float32)]
    scratch = [
        pltpu.VMEM((NIDX, 2, CHUNK), jnp.int32),
        pltpu.VMEM((NBUF, CHUNK, DH), jnp.float32),
        pltpu.SemaphoreType.DMA,
        pltpu.SemaphoreType.DMA,
        pltpu.SemaphoreType.DMA,
        pltpu.VMEM_SHARED((N_PAD, DH), jnp.float32),
        pltpu.VMEM_SHARED((N, DH), jnp.float32),
    ]
    if with_cnt:
        outs.append(jax.ShapeDtypeStruct((NC, N_PAD, CW), jnp.float32))
        scratch += [
            pltpu.VMEM((CHUNK, CW), jnp.float32),
            pltpu.SemaphoreType.DMA,
            pltpu.VMEM_SHARED((N_PAD, CW), jnp.float32),
        ]
    fn = pl.kernel(
        functools.partial(_seg_sum_body, with_cnt),
        out_type=tuple(outs),
        mesh=_MESH,
        scratch_types=scratch,
        compiler_params=pltpu.CompilerParams(use_tc_tiling_on_sc=False),
    )
    return fn(hs, idx)


_RB = 400
_GRID = N // _RB


def _linear_body(x_ref, w_ref, b_ref, o_ref):
    z = lax.dot_general(
        x_ref[...], w_ref[...], (((1,), (1,)), ((), ())),
        preferred_element_type=jnp.float32) + b_ref[...]
    o_ref[...] = jnp.stack([z[:, :DH], z[:, DH:]])


def _linear_split(x, w, b):
    return pl.pallas_call(
        _linear_body,
        grid=(_GRID,),
        in_specs=[
            pl.BlockSpec((_RB, D), lambda i: (i, 0)),
            pl.BlockSpec((D, D), lambda i: (0, 0)),
            pl.BlockSpec((1, D), lambda i: (0, 0)),
        ],
        out_specs=pl.BlockSpec((NC, _RB, DH), lambda i: (0, i, 0)),
        out_shape=jax.ShapeDtypeStruct((NC, N, DH), jnp.float32),
    )(x, w, b.reshape(1, D))


def _sage_body(mode, p_ref, c_ref, h_ref, wl_ref, bl_ref, wr_ref, o_ref):
    cnt = jnp.maximum(c_ref[0][:, 0:1], 1.0)
    mean = jnp.concatenate([p_ref[0], p_ref[1]], axis=1) / cnt
    h = jnp.concatenate([h_ref[0], h_ref[1]], axis=1)
    z = lax.dot_general(mean, wl_ref[...], (((1,), (1,)), ((), ())),
                        preferred_element_type=jnp.float32)
    z = z + lax.dot_general(h, wr_ref[...], (((1,), (1,)), ((), ())),
                            preferred_element_type=jnp.float32)
    z = z + bl_ref[...]
    if mode == "relu":
        z = jnp.maximum(z, 0.0)
        o_ref[...] = jnp.stack([z[:, :DH], z[:, DH:]])
    else:
        nrm = jnp.sqrt(jnp.sum(z * z, axis=1, keepdims=True))
        o_ref[...] = z / jnp.maximum(nrm, 1e-12)


def _sage_tc(p, cnt, hs, wl, bl, wr, mode):
    if mode == "relu":
        out_spec = pl.BlockSpec((NC, _RB, DH), lambda i: (0, i, 0))
        out_shape = jax.ShapeDtypeStruct((NC, N, DH), jnp.float32)
    else:
        out_spec = pl.BlockSpec((_RB, D), lambda i: (i, 0))
        out_shape = jax.ShapeDtypeStruct((N, D), jnp.float32)
    return pl.pallas_call(
        functools.partial(_sage_body, mode),
        grid=(_GRID,),
        in_specs=[
            pl.BlockSpec((NC, _RB, DH), lambda i: (0, i, 0)),
            pl.BlockSpec((NC, _RB, CW), lambda i: (0, i, 0)),
            pl.BlockSpec((NC, _RB, DH), lambda i: (0, i, 0)),
            pl.BlockSpec((D, D), lambda i: (0, 0)),
            pl.BlockSpec((1, D), lambda i: (0, 0)),
            pl.BlockSpec((D, D), lambda i: (0, 0)),
        ],
        out_specs=out_spec,
        out_shape=out_shape,
    )(p, cnt, hs, wl, bl.reshape(1, D), wr)


def kernel(x, edge_index, W_pre, b_pre, Wl1, bl1, Wr1, Wl2, bl2, Wr2,
           Wl3, bl3, Wr3):
    src = edge_index[0].astype(jnp.int32)
    dst = edge_index[1].astype(jnp.int32)
    pad = E_PAD - E
    sp3 = jnp.concatenate(
        [src, jnp.zeros((pad,), jnp.int32)]).reshape(NS, CPT, CHUNK)
    dst3 = jnp.concatenate(
        [dst, jnp.full((pad,), TRASH, jnp.int32)]).reshape(NS, CPT, CHUNK)
    idx = jnp.stack([sp3, dst3], axis=2)

    h0s = _linear_split(x, W_pre, b_pre)
    p1, cnt = _seg_sum(h0s, idx, with_cnt=True)
    h1s = _sage_tc(p1, cnt, h0s, Wl1, bl1, Wr1, "relu")
    p2, = _seg_sum(h1s, idx, with_cnt=False)
    h2s = _sage_tc(p2, cnt, h1s, Wl2, bl2, Wr2, "relu")
    p3, = _seg_sum(h2s, idx, with_cnt=False)
    return _sage_tc(p3, cnt, h2s, Wl3, bl3, Wr3, "norm")


# speedup vs baseline: 10.1609x; 1.1311x over previous
import functools

import jax
import jax.numpy as jnp
from jax import lax
from jax.experimental import pallas as pl
from jax.experimental.pallas import tpu as pltpu
from jax.experimental.pallas import tpu_sc as plsc

N = 10000
D = 128
DH = D // 2
E = 320000

NC = 2
NS = 16
CHUNK = 128
CPT = 160
EPT = CPT * CHUNK
E_PAD = NS * EPT
N_PAD = 10240
RPT = N_PAD // NS
KPT = RPT // CHUNK
HPT = N // NS
TRASH = N
CW = 16
NBUF = 4
DEPTH = NBUF // 2
NIDX = 8

_MESH = plsc.VectorSubcoreMesh(core_axis_name="c", subcore_axis_name="s")


def _seg_sum_body(with_cnt, hs_hbm, idx_hbm, *rest):
    if with_cnt:
        (out_p, out_c, sidx, rows, sem_i, sem_g, sem_s, acc, h_sp,
         ones_v, sem_c, accc) = rest
    else:
        out_p, sidx, rows, sem_i, sem_g, sem_s, acc, h_sp = rest
    c = lax.axis_index("c")
    s = lax.axis_index("s")

    pltpu.sync_copy(hs_hbm.at[c].at[pl.ds(s * HPT, HPT)],
                    h_sp.at[pl.ds(s * HPT, HPT)])

    @pl.loop(0, CHUNK)
    def _(r):
        @pl.loop(0, DH, step=16)
        def _(cc):
            rows.at[0, r, pl.ds(cc, 16)][...] = jnp.zeros((16,), jnp.float32)

    for k in range(KPT):
        pltpu.sync_copy(rows.at[0],
                        acc.at[pl.ds(s * RPT + k * CHUNK, CHUNK)])

    if with_cnt:
        @pl.loop(0, CHUNK)
        def _(r):
            ones_v.at[r, pl.ds(0, CW)][...] = jnp.zeros((CW,), jnp.float32)

        for k in range(KPT):
            pltpu.sync_copy(
                ones_v, accc.at[pl.ds(s * RPT + k * CHUNK, CHUNK)])

        @pl.loop(0, CHUNK)
        def _(r):
            ones_v.at[r, pl.ds(0, CW)][...] = jnp.ones((CW,), jnp.float32)

    plsc.subcore_barrier()

    def idx_start(j, ib):
        pltpu.async_copy(idx_hbm.at[s].at[j], sidx.at[ib], sem_i)

    def idx_wait(j, ib):
        pltpu.make_async_copy(idx_hbm.at[s].at[j], sidx.at[ib], sem_i).wait()

    def gather_start(ib, rb):
        pltpu.async_copy(h_sp.at[sidx.at[ib].at[0]], rows.at[rb], sem_g)

    def gather_wait(ib, rb):
        pltpu.make_async_copy(
            h_sp.at[sidx.at[ib].at[0]], rows.at[rb], sem_g).wait()

    def scat_start(ib, rb):
        pltpu.async_copy(rows.at[rb], acc.at[sidx.at[ib].at[1]], sem_s,
                         add=True)

    def scat_wait(ib, rb):
        pltpu.make_async_copy(
            rows.at[rb], acc.at[sidx.at[ib].at[1]], sem_s).wait()

    def cnt_start(ib):
        pltpu.async_copy(ones_v, accc.at[sidx.at[ib].at[1]], sem_c, add=True)

    def cnt_wait(ib):
        pltpu.make_async_copy(ones_v, accc.at[sidx.at[ib].at[1]], sem_c).wait()

    for t in range(DEPTH + 1):
        idx_start(t, t % NIDX)
    for b in range(DEPTH):
        idx_wait(b, b % NIDX)
        gather_start(b % NIDX, b % NBUF)

    @pl.loop(0, CPT, step=NIDX)
    def _(g):
        for b in range(NIDX):
            j = g + b
            gather_wait(b % NIDX, b % NBUF)
            scat_start(b % NIDX, b % NBUF)
            if with_cnt:
                @pl.when(c == b % 2)
                def _():
                    cnt_start(b % NIDX)

                @pl.when(jnp.logical_and(c == b % 2, j >= DEPTH))
                def _():
                    cnt_wait(b % NIDX)

            @pl.when(j >= DEPTH)
            def _():
                scat_wait((b - DEPTH) % NIDX, (b - DEPTH) % NBUF)

            @pl.when(j + DEPTH < CPT)
            def _():
                idx_wait(j + DEPTH, (b + DEPTH) % NIDX)
                gather_start((b + DEPTH) % NIDX, (b + DEPTH) % NBUF)

            @pl.when(j + DEPTH + 1 < CPT)
            def _():
                idx_start(j + DEPTH + 1, (b + DEPTH + 1) % NIDX)

    for b in range(DEPTH):
        j = CPT - DEPTH + b
        scat_wait(j % NIDX, j % NBUF)
        if with_cnt:
            @pl.when(c == j % 2)
            def _():
                cnt_wait(j % NIDX)

    plsc.subcore_barrier()

    pltpu.sync_copy(acc.at[pl.ds(s * RPT, RPT)],
                    out_p.at[c].at[pl.ds(s * RPT, RPT)])
    if with_cnt:
        pltpu.sync_copy(accc.at[pl.ds(s * RPT, RPT)],
                        out_c.at[c].at[pl.ds(s * RPT, RPT)])


def _seg_sum(hs, idx, with_cnt):
    outs = [jax.ShapeDtypeStruct((NC, N_PAD, DH), jnp.float32)]
    scratch = [
        pltpu.VMEM((NIDX, 2, CHUNK), jnp.int32),
        pltpu.VMEM((NBUF, CHUNK, DH), jnp.float32),
        pltpu.SemaphoreType.DMA,
        pltpu.SemaphoreType.DMA,
        pltpu.SemaphoreType.DMA,
        pltpu.VMEM_SHARED((N_PAD, DH), jnp.float32),
        pltpu.VMEM_SHARED((N, DH), jnp.float32),
    ]
    if with_cnt:
        outs.append(jax.ShapeDtypeStruct((NC, N_PAD, CW), jnp.float32))
        scratch += [
            pltpu.VMEM((CHUNK, CW), jnp.float32),
            pltpu.SemaphoreType.DMA,
            pltpu.VMEM_SHARED((N_PAD, CW), jnp.float32),
        ]
    fn = pl.kernel(
        functools.partial(_seg_sum_body, with_cnt),
        out_type=tuple(outs),
        mesh=_MESH,
        scratch_types=scratch,
        compiler_params=pltpu.CompilerParams(use_tc_tiling_on_sc=False),
    )
    return fn(hs, idx)


_RB = 2000
_GRID = N // _RB


def _linear_body(x_ref, w_ref, b_ref, o_ref):
    z = lax.dot_general(
        x_ref[...], w_ref[...], (((1,), (1,)), ((), ())),
        preferred_element_type=jnp.float32) + b_ref[...]
    o_ref[...] = jnp.stack([z[:, :DH], z[:, DH:]])


def _linear_split(x, w, b):
    return pl.pallas_call(
        _linear_body,
        grid=(_GRID,),
        in_specs=[
            pl.BlockSpec((_RB, D), lambda i: (i, 0)),
            pl.BlockSpec((D, D), lambda i: (0, 0)),
            pl.BlockSpec((1, D), lambda i: (0, 0)),
        ],
        out_specs=pl.BlockSpec((NC, _RB, DH), lambda i: (0, i, 0)),
        out_shape=jax.ShapeDtypeStruct((NC, N, DH), jnp.float32),
    )(x, w, b.reshape(1, D))


def _sage_body(mode, p_ref, c_ref, h_ref, wl_ref, bl_ref, wr_ref, o_ref):
    cnt = jnp.maximum(c_ref[0][:, 0:1] + c_ref[1][:, 0:1], 1.0)
    mean = jnp.concatenate([p_ref[0], p_ref[1]], axis=1) / cnt
    h = jnp.concatenate([h_ref[0], h_ref[1]], axis=1)
    z = lax.dot_general(mean, wl_ref[...], (((1,), (1,)), ((), ())),
                        preferred_element_type=jnp.float32)
    z = z + lax.dot_general(h, wr_ref[...], (((1,), (1,)), ((), ())),
                            preferred_element_type=jnp.float32)
    z = z + bl_ref[...]
    if mode == "relu":
        z = jnp.maximum(z, 0.0)
        o_ref[...] = jnp.stack([z[:, :DH], z[:, DH:]])
    else:
        nrm = jnp.sqrt(jnp.sum(z * z, axis=1, keepdims=True))
        o_ref[...] = z / jnp.maximum(nrm, 1e-12)


def _sage_tc(p, cnt, hs, wl, bl, wr, mode):
    if mode == "relu":
        out_spec = pl.BlockSpec((NC, _RB, DH), lambda i: (0, i, 0))
        out_shape = jax.ShapeDtypeStruct((NC, N, DH), jnp.float32)
    else:
        out_spec = pl.BlockSpec((_RB, D), lambda i: (i, 0))
        out_shape = jax.ShapeDtypeStruct((N, D), jnp.float32)
    return pl.pallas_call(
        functools.partial(_sage_body, mode),
        grid=(_GRID,),
        in_specs=[
            pl.BlockSpec((NC, _RB, DH), lambda i: (0, i, 0)),
            pl.BlockSpec((NC, _RB, CW), lambda i: (0, i, 0)),
            pl.BlockSpec((NC, _RB, DH), lambda i: (0, i, 0)),
            pl.BlockSpec((D, D), lambda i: (0, 0)),
            pl.BlockSpec((1, D), lambda i: (0, 0)),
            pl.BlockSpec((D, D), lambda i: (0, 0)),
        ],
        out_specs=out_spec,
        out_shape=out_shape,
    )(p, cnt, hs, wl, bl.reshape(1, D), wr)


def kernel(x, edge_index, W_pre, b_pre, Wl1, bl1, Wr1, Wl2, bl2, Wr2,
           Wl3, bl3, Wr3):
    src = edge_index[0].astype(jnp.int32)
    dst = edge_index[1].astype(jnp.int32)
    pad = E_PAD - E
    sp3 = jnp.concatenate(
        [src, jnp.zeros((pad,), jnp.int32)]).reshape(NS, CPT, CHUNK)
    dst3 = jnp.concatenate(
        [dst, jnp.full((pad,), TRASH, jnp.int32)]).reshape(NS, CPT, CHUNK)
    idx = jnp.stack([sp3, dst3], axis=2)

    h0s = _linear_split(x, W_pre, b_pre)
    p1, cnt = _seg_sum(h0s, idx, with_cnt=True)
    h1s = _sage_tc(p1, cnt, h0s, Wl1, bl1, Wr1, "relu")
    p2, = _seg_sum(h1s, idx, with_cnt=False)
    h2s = _sage_tc(p2, cnt, h1s, Wl2, bl2, Wr2, "relu")
    p3, = _seg_sum(h2s, idx, with_cnt=False)
    return _sage_tc(p3, cnt, h2s, Wl3, bl3, Wr3, "norm")
